# Initial kernel scaffold; baseline (speedup 1.0000x reference)
import jax
import jax.numpy as jnp
from jax.experimental import pallas as pl


def kernel(h_V, h_E, E_idx, mask_V, mask_attend, params):
    raise NotImplementedError("write your pallas kernel here")


# speedup vs baseline: 4.6077x; 4.6077x over previous
import functools
import math

import jax
import jax.numpy as jnp
from jax import lax
from jax.experimental import pallas as pl
from jax.experimental.pallas import tpu as pltpu
from jax.experimental.pallas import tpu_sc as plsc

H = 128
K = 16
SCALE = 30.0
EPS = 1e-5
TN = 200
CH = 128

_SQRT2 = math.sqrt(2.0)


def _gelu(x):
    return 0.5 * x * (1.0 + lax.erf(x / _SQRT2))


def _layernorm(x, g, b):
    m = jnp.mean(x, axis=-1, keepdims=True)
    v = jnp.mean((x - m) ** 2, axis=-1, keepdims=True)
    return (x - m) * lax.rsqrt(v + EPS) * g + b


def _dot(a, b):
    return jnp.dot(a, b, preferred_element_type=jnp.float32)


def _sc_gather(table, idx):
    nkpad = idx.shape[0]
    info = plsc.get_sparse_core_info()
    nc, ns = info.num_cores, info.num_subcores
    nw = nc * ns
    per_w = nkpad // nw
    assert per_w % CH == 0
    nch = per_w // CH
    mesh = plsc.VectorSubcoreMesh(core_axis_name="c", subcore_axis_name="s")

    @functools.partial(
        pl.kernel,
        mesh=mesh,
        out_type=jax.ShapeDtypeStruct((nkpad, H), jnp.float32),
        scratch_types=[
            pltpu.VMEM((CH,), jnp.int32),
            pltpu.VMEM((CH, H), jnp.float32),
            pltpu.SemaphoreType.DMA,
        ],
    )
    def gk(table_hbm, idx_hbm, out_hbm, idx_v, rows_v, sem):
        wid = lax.axis_index("s") * nc + lax.axis_index("c")
        base = wid * per_w

        def body(i, carry):
            off = base + i * CH
            pltpu.sync_copy(idx_hbm.at[pl.ds(off, CH)], idx_v)
            pltpu.async_copy(table_hbm.at[idx_v], rows_v, sem).wait()
            pltpu.sync_copy(rows_v, out_hbm.at[pl.ds(off, CH)])
            return carry

        lax.fori_loop(0, nch, body, 0)

    return gk(table, idx)


def _c1_body(hv_ref, w_ref, out_ref):
    out_ref[...] = _dot(hv_ref[...], w_ref[...])


def _node_body(hv_ref, he_ref, g1_ref, ma_ref, mv_ref,
               w1a_ref, b1_ref, w1b_ref, w2_ref, b2_ref, w3_ref, b3_ref,
               win_ref, bin_ref, wout_ref, bout_ref,
               ln1g_ref, ln1b_ref, ln2g_ref, ln2b_ref, w11c_ref,
               hv2_ref, c2_ref):
    hv = hv_ref[...]
    a1 = _dot(hv, w1a_ref[...]) + b1_ref[...]
    a1e = jnp.reshape(
        jnp.broadcast_to(a1[:, None, :], (TN, K, H)), (TN * K, H))
    x = _dot(he_ref[...], w1b_ref[...]) + g1_ref[...] + a1e
    x = _gelu(x)
    x = _gelu(_dot(x, w2_ref[...]) + b2_ref[...])
    m = _dot(x, w3_ref[...]) + b3_ref[...]
    m = m * ma_ref[...]
    dh = jnp.sum(jnp.reshape(m, (TN, K, H)), axis=1) * (1.0 / SCALE)
    h = _layernorm(hv + dh, ln1g_ref[...], ln1b_ref[...])
    f = _gelu(_dot(h, win_ref[...]) + bin_ref[...])
    f = _dot(f, wout_ref[...]) + bout_ref[...]
    h2 = _layernorm(h + f, ln2g_ref[...], ln2b_ref[...]) * mv_ref[...]
    hv2_ref[...] = h2
    c2_ref[...] = _dot(h2, w11c_ref[...])


def _edge_body(hv_ref, he_ref, g2_ref,
               w11a_ref, b11_ref, w11b_ref, w12_ref, b12_ref,
               w13_ref, b13_ref, ln3g_ref, ln3b_ref,
               heo_ref):
    a1 = _dot(hv_ref[...], w11a_ref[...]) + b11_ref[...]
    a1e = jnp.reshape(
        jnp.broadcast_to(a1[:, None, :], (TN, K, H)), (TN * K, H))
    he = he_ref[...]
    y = _dot(he, w11b_ref[...]) + g2_ref[...] + a1e
    y = _gelu(y)
    y = _gelu(_dot(y, w12_ref[...]) + b12_ref[...])
    m = _dot(y, w13_ref[...]) + b13_ref[...]
    heo_ref[...] = _layernorm(he + m, ln3g_ref[...], ln3b_ref[...])


def _full(shape):
    return pl.BlockSpec(shape, lambda i: (0,) * len(shape))


def kernel(h_V, h_E, E_idx, mask_V, mask_attend, params):
    p = params
    n = h_V.shape[1]
    nk = n * K
    hv = h_V[0]
    he = jnp.reshape(h_E[0], (nk, H))
    idx = jnp.reshape(E_idx[0], (nk,)).astype(jnp.int32)
    ma = jnp.reshape(mask_attend[0], (nk, 1))
    mv = jnp.reshape(mask_V[0], (n, 1))

    info = plsc.get_sparse_core_info()
    nw = info.num_cores * info.num_subcores
    quanta = nw * CH
    nkpad = ((nk + quanta - 1) // quanta) * quanta
    idx_pad = jnp.concatenate(
        [idx, jnp.zeros((nkpad - nk,), jnp.int32)]) if nkpad != nk else idx

    w1t = p['W1'].T
    w1a, w1b, w1c = w1t[:H], w1t[H:2 * H], w1t[2 * H:]
    w11t = p['W11'].T
    w11a, w11b, w11c = w11t[:H], w11t[H:2 * H], w11t[2 * H:]
    w2, w3 = p['W2'].T, p['W3'].T
    w12, w13 = p['W12'].T, p['W13'].T
    win, wout = p['Win'].T, p['Wout'].T
    row = lambda v: jnp.reshape(v, (1, -1))

    grid = (n // TN,)
    node_spec = pl.BlockSpec((TN, H), lambda i: (i, 0))
    edge_spec = pl.BlockSpec((TN * K, H), lambda i: (i, 0))

    c1 = pl.pallas_call(
        _c1_body,
        grid=(n // 1000,),
        in_specs=[pl.BlockSpec((1000, H), lambda i: (i, 0)), _full((H, H))],
        out_specs=pl.BlockSpec((1000, H), lambda i: (i, 0)),
        out_shape=jax.ShapeDtypeStruct((n, H), jnp.float32),
    )(hv, w1c)

    g1 = _sc_gather(c1, idx_pad)

    hv2, c2 = pl.pallas_call(
        _node_body,
        grid=grid,
        in_specs=[
            node_spec, edge_spec, edge_spec,
            pl.BlockSpec((TN * K, 1), lambda i: (i, 0)),
            pl.BlockSpec((TN, 1), lambda i: (i, 0)),
            _full((H, H)), _full((1, H)), _full((H, H)),
            _full((H, H)), _full((1, H)), _full((H, H)), _full((1, H)),
            _full((H, 4 * H)), _full((1, 4 * H)), _full((4 * H, H)),
            _full((1, H)),
            _full((1, H)), _full((1, H)), _full((1, H)), _full((1, H)),
            _full((H, H)),
        ],
        out_specs=[node_spec, node_spec],
        out_shape=[jax.ShapeDtypeStruct((n, H), jnp.float32),
                   jax.ShapeDtypeStruct((n, H), jnp.float32)],
    )(hv, he, g1, ma, mv,
      w1a, row(p['b1']), w1b, w2, row(p['b2']), w3, row(p['b3']),
      win, row(p['bin']), wout, row(p['bout']),
      row(p['ln1_g']), row(p['ln1_b']), row(p['ln2_g']), row(p['ln2_b']),
      w11c)

    g2 = _sc_gather(c2, idx_pad)

    heo = pl.pallas_call(
        _edge_body,
        grid=grid,
        in_specs=[
            node_spec, edge_spec, edge_spec,
            _full((H, H)), _full((1, H)), _full((H, H)),
            _full((H, H)), _full((1, H)), _full((H, H)), _full((1, H)),
            _full((1, H)), _full((1, H)),
        ],
        out_specs=edge_spec,
        out_shape=jax.ShapeDtypeStruct((nk, H), jnp.float32),
    )(hv2, he, g2,
      w11a, row(p['b11']), w11b, w12, row(p['b12']), w13, row(p['b13']),
      row(p['ln3_g']), row(p['ln3_b']))

    return (hv2[None], jnp.reshape(heo, (1, n, K, H)))


# speedup vs baseline: 4.9791x; 1.0806x over previous
import functools
import math

import jax
import jax.numpy as jnp
from jax import lax
from jax.experimental import pallas as pl
from jax.experimental.pallas import tpu as pltpu
from jax.experimental.pallas import tpu_sc as plsc

H = 128
K = 16
SCALE = 30.0
EPS = 1e-5
TN = 200
CH = 128

_SQRT2 = math.sqrt(2.0)


def _gelu(x):
    return 0.5 * x * (1.0 + lax.erf(x / _SQRT2))


def _layernorm(x, g, b):
    m = jnp.mean(x, axis=-1, keepdims=True)
    v = jnp.mean((x - m) ** 2, axis=-1, keepdims=True)
    return (x - m) * lax.rsqrt(v + EPS) * g + b


def _dot(a, b):
    return jnp.dot(a, b, preferred_element_type=jnp.float32)


def _sc_gather(table, idx):
    nkpad = idx.shape[0]
    info = plsc.get_sparse_core_info()
    nc, ns = info.num_cores, info.num_subcores
    nw = nc * ns
    per_w = nkpad // nw
    assert per_w % CH == 0
    nch = per_w // CH
    mesh = plsc.VectorSubcoreMesh(core_axis_name="c", subcore_axis_name="s")

    @functools.partial(
        pl.kernel,
        mesh=mesh,
        out_type=jax.ShapeDtypeStruct((nkpad, H), jnp.float32),
        scratch_types=[
            pltpu.VMEM((per_w,), jnp.int32),
            pltpu.VMEM((CH, H), jnp.float32),
            pltpu.VMEM((CH, H), jnp.float32),
            pltpu.SemaphoreType.DMA,
            pltpu.SemaphoreType.DMA,
            pltpu.SemaphoreType.DMA,
            pltpu.SemaphoreType.DMA,
        ],
    )
    def gk(table_hbm, idx_hbm, out_hbm, idx_v, rows0, rows1, sg0, sg1,
           sw0, sw1):
        wid = lax.axis_index("s") * nc + lax.axis_index("c")
        base = wid * per_w
        rows = (rows0, rows1)
        sg = (sg0, sg1)
        sw = (sw0, sw1)

        pltpu.sync_copy(idx_hbm.at[pl.ds(base, per_w)], idx_v)

        def gather_start(c, b):
            pltpu.async_copy(
                table_hbm.at[idx_v.at[pl.ds(c * CH, CH)]], rows[b], sg[b])

        def wb_start(c, b):
            pltpu.async_copy(
                rows[b], out_hbm.at[pl.ds(base + c * CH, CH)], sw[b])

        def wb_wait(c, b):
            pltpu.make_async_copy(
                rows[b], out_hbm.at[pl.ds(base + c * CH, CH)], sw[b]).wait()

        gather_start(0, 0)

        def body(g, carry):
            for b in range(2):
                c = g + b
                nb = 1 - b
                pltpu.make_async_copy(
                    table_hbm.at[idx_v.at[pl.ds(c * CH, CH)]],
                    rows[b], sg[b]).wait()

                @pl.when(c + 1 < nch)
                def _():
                    @pl.when(c >= 1)
                    def _():
                        wb_wait(c - 1, nb)
                    gather_start(c + 1, nb)

                wb_start(c, b)
            return carry

        lax.fori_loop(0, nch // 2, lambda g, c: body(g * 2, c), 0)
        wb_wait(nch - 2, 0)
        wb_wait(nch - 1, 1)

    return gk(table, idx)


def _c1_body(hv_ref, w_ref, out_ref):
    out_ref[...] = _dot(hv_ref[...], w_ref[...])


def _node_body(hv_ref, he_ref, g1_ref, ma_ref, mv_ref,
               w1a_ref, b1_ref, w1b_ref, w2_ref, b2_ref, w3_ref, b3_ref,
               win_ref, bin_ref, wout_ref, bout_ref,
               ln1g_ref, ln1b_ref, ln2g_ref, ln2b_ref, w11c_ref,
               hv2_ref, c2_ref):
    hv = hv_ref[...]
    a1 = _dot(hv, w1a_ref[...]) + b1_ref[...]
    a1e = jnp.reshape(
        jnp.broadcast_to(a1[:, None, :], (TN, K, H)), (TN * K, H))
    x = _dot(he_ref[...], w1b_ref[...]) + g1_ref[...] + a1e
    x = _gelu(x)
    x = _gelu(_dot(x, w2_ref[...]) + b2_ref[...])
    m = _dot(x, w3_ref[...]) + b3_ref[...]
    m = m * ma_ref[...]
    dh = jnp.sum(jnp.reshape(m, (TN, K, H)), axis=1) * (1.0 / SCALE)
    h = _layernorm(hv + dh, ln1g_ref[...], ln1b_ref[...])
    f = _gelu(_dot(h, win_ref[...]) + bin_ref[...])
    f = _dot(f, wout_ref[...]) + bout_ref[...]
    h2 = _layernorm(h + f, ln2g_ref[...], ln2b_ref[...]) * mv_ref[...]
    hv2_ref[...] = h2
    c2_ref[...] = _dot(h2, w11c_ref[...])


def _edge_body(hv_ref, he_ref, g2_ref,
               w11a_ref, b11_ref, w11b_ref, w12_ref, b12_ref,
               w13_ref, b13_ref, ln3g_ref, ln3b_ref,
               heo_ref):
    a1 = _dot(hv_ref[...], w11a_ref[...]) + b11_ref[...]
    a1e = jnp.reshape(
        jnp.broadcast_to(a1[:, None, :], (TN, K, H)), (TN * K, H))
    he = he_ref[...]
    y = _dot(he, w11b_ref[...]) + g2_ref[...] + a1e
    y = _gelu(y)
    y = _gelu(_dot(y, w12_ref[...]) + b12_ref[...])
    m = _dot(y, w13_ref[...]) + b13_ref[...]
    heo_ref[...] = _layernorm(he + m, ln3g_ref[...], ln3b_ref[...])


def _full(shape):
    return pl.BlockSpec(shape, lambda i: (0,) * len(shape))


def kernel(h_V, h_E, E_idx, mask_V, mask_attend, params):
    p = params
    n = h_V.shape[1]
    nk = n * K
    hv = h_V[0]
    he = jnp.reshape(h_E[0], (nk, H))
    idx = jnp.reshape(E_idx[0], (nk,)).astype(jnp.int32)
    ma = jnp.reshape(mask_attend[0], (nk, 1))
    mv = jnp.reshape(mask_V[0], (n, 1))

    info = plsc.get_sparse_core_info()
    nw = info.num_cores * info.num_subcores
    quanta = nw * CH
    nkpad = ((nk + quanta - 1) // quanta) * quanta
    idx_pad = jnp.concatenate(
        [idx, jnp.zeros((nkpad - nk,), jnp.int32)]) if nkpad != nk else idx

    w1t = p['W1'].T
    w1a, w1b, w1c = w1t[:H], w1t[H:2 * H], w1t[2 * H:]
    w11t = p['W11'].T
    w11a, w11b, w11c = w11t[:H], w11t[H:2 * H], w11t[2 * H:]
    w2, w3 = p['W2'].T, p['W3'].T
    w12, w13 = p['W12'].T, p['W13'].T
    win, wout = p['Win'].T, p['Wout'].T
    row = lambda v: jnp.reshape(v, (1, -1))

    grid = (n // TN,)
    node_spec = pl.BlockSpec((TN, H), lambda i: (i, 0))
    edge_spec = pl.BlockSpec((TN * K, H), lambda i: (i, 0))

    c1 = pl.pallas_call(
        _c1_body,
        grid=(n // 1000,),
        in_specs=[pl.BlockSpec((1000, H), lambda i: (i, 0)), _full((H, H))],
        out_specs=pl.BlockSpec((1000, H), lambda i: (i, 0)),
        out_shape=jax.ShapeDtypeStruct((n, H), jnp.float32),
    )(hv, w1c)

    g1 = _sc_gather(c1, idx_pad)

    hv2, c2 = pl.pallas_call(
        _node_body,
        grid=grid,
        in_specs=[
            node_spec, edge_spec, edge_spec,
            pl.BlockSpec((TN * K, 1), lambda i: (i, 0)),
            pl.BlockSpec((TN, 1), lambda i: (i, 0)),
            _full((H, H)), _full((1, H)), _full((H, H)),
            _full((H, H)), _full((1, H)), _full((H, H)), _full((1, H)),
            _full((H, 4 * H)), _full((1, 4 * H)), _full((4 * H, H)),
            _full((1, H)),
            _full((1, H)), _full((1, H)), _full((1, H)), _full((1, H)),
            _full((H, H)),
        ],
        out_specs=[node_spec, node_spec],
        out_shape=[jax.ShapeDtypeStruct((n, H), jnp.float32),
                   jax.ShapeDtypeStruct((n, H), jnp.float32)],
    )(hv, he, g1, ma, mv,
      w1a, row(p['b1']), w1b, w2, row(p['b2']), w3, row(p['b3']),
      win, row(p['bin']), wout, row(p['bout']),
      row(p['ln1_g']), row(p['ln1_b']), row(p['ln2_g']), row(p['ln2_b']),
      w11c)

    g2 = _sc_gather(c2, idx_pad)

    heo = pl.pallas_call(
        _edge_body,
        grid=grid,
        in_specs=[
            node_spec, edge_spec, edge_spec,
            _full((H, H)), _full((1, H)), _full((H, H)),
            _full((H, H)), _full((1, H)), _full((H, H)), _full((1, H)),
            _full((1, H)), _full((1, H)),
        ],
        out_specs=edge_spec,
        out_shape=jax.ShapeDtypeStruct((nk, H), jnp.float32),
    )(hv2, he, g2,
      w11a, row(p['b11']), w11b, w12, row(p['b12']), w13, row(p['b13']),
      row(p['ln3_g']), row(p['ln3_b']))

    return (hv2[None], jnp.reshape(heo, (1, n, K, H)))


# speedup vs baseline: 5.5223x; 1.1091x over previous
import functools
import math

import jax
import jax.numpy as jnp
from jax import lax
from jax.experimental import pallas as pl
from jax.experimental.pallas import tpu as pltpu
from jax.experimental.pallas import tpu_sc as plsc

H = 128
K = 16
SCALE = 30.0
EPS = 1e-5
TN = 400
CH = 128

_SQRT2 = math.sqrt(2.0)


def _gelu(x):
    return 0.5 * x * (1.0 + lax.erf(x / _SQRT2))


def _layernorm(x, g, b):
    m = jnp.mean(x, axis=-1, keepdims=True)
    v = jnp.mean((x - m) ** 2, axis=-1, keepdims=True)
    return (x - m) * lax.rsqrt(v + EPS) * g + b


def _dot(a, b):
    return jnp.dot(a, b, preferred_element_type=jnp.float32)


def _sc_gather(table, idx):
    nkpad = idx.shape[0]
    info = plsc.get_sparse_core_info()
    nc, ns = info.num_cores, info.num_subcores
    nw = nc * ns
    per_w = nkpad // nw
    assert per_w % CH == 0
    nch = per_w // CH
    mesh = plsc.VectorSubcoreMesh(core_axis_name="c", subcore_axis_name="s")

    nbuf = 4
    assert nch % nbuf == 0

    @functools.partial(
        pl.kernel,
        mesh=mesh,
        out_type=jax.ShapeDtypeStruct((nkpad, H), jnp.float32),
        scratch_types=[
            pltpu.VMEM((per_w,), jnp.int32),
            [pltpu.VMEM((CH, H), jnp.float32) for _ in range(nbuf)],
            [pltpu.SemaphoreType.DMA for _ in range(nbuf)],
            [pltpu.SemaphoreType.DMA for _ in range(nbuf)],
        ],
    )
    def gk(table_hbm, idx_hbm, out_hbm, idx_v, rows, sg, sw):
        wid = lax.axis_index("s") * nc + lax.axis_index("c")
        base = wid * per_w

        pltpu.sync_copy(idx_hbm.at[pl.ds(base, per_w)], idx_v)

        def gather_start(c, b):
            pltpu.async_copy(
                table_hbm.at[idx_v.at[pl.ds(c * CH, CH)]], rows[b], sg[b])

        def gather_wait(c, b):
            pltpu.make_async_copy(
                table_hbm.at[idx_v.at[pl.ds(c * CH, CH)]],
                rows[b], sg[b]).wait()

        def wb_start(c, b):
            pltpu.async_copy(
                rows[b], out_hbm.at[pl.ds(base + c * CH, CH)], sw[b])

        def wb_wait(c, b):
            pltpu.make_async_copy(
                rows[b], out_hbm.at[pl.ds(base + c * CH, CH)], sw[b]).wait()

        gather_start(0, 0)
        gather_start(1, 1)

        def body(g, carry):
            for b in range(nbuf):
                c = g + b
                b2 = (b + 2) % nbuf
                gather_wait(c, b)

                @pl.when(c + 2 < nch)
                def _():
                    @pl.when(c >= 2)
                    def _():
                        wb_wait(c - 2, b2)
                    gather_start(c + 2, b2)

                wb_start(c, b)
            return carry

        lax.fori_loop(0, nch // nbuf, lambda g, c: body(g * nbuf, c), 0)
        for j in range(nbuf):
            c = nch - nbuf + j
            wb_wait(c, c % nbuf)

    return gk(table, idx)


def _c1_body(hv_ref, w_ref, out_ref):
    out_ref[...] = _dot(hv_ref[...], w_ref[...])


def _node_body(hv_ref, he_ref, g1_ref, ma_ref, mv_ref,
               w1a_ref, b1_ref, w1b_ref, w2_ref, b2_ref, w3_ref, b3_ref,
               win_ref, bin_ref, wout_ref, bout_ref,
               ln1g_ref, ln1b_ref, ln2g_ref, ln2b_ref, w11c_ref,
               hv2_ref, c2_ref):
    hv = hv_ref[...]
    a1 = _dot(hv, w1a_ref[...]) + b1_ref[...]
    a1e = jnp.reshape(
        jnp.broadcast_to(a1[:, None, :], (TN, K, H)), (TN * K, H))
    x = _dot(he_ref[...], w1b_ref[...]) + g1_ref[...] + a1e
    x = _gelu(x)
    x = _gelu(_dot(x, w2_ref[...]) + b2_ref[...])
    m = _dot(x, w3_ref[...]) + b3_ref[...]
    m = m * ma_ref[...]
    dh = jnp.sum(jnp.reshape(m, (TN, K, H)), axis=1) * (1.0 / SCALE)
    h = _layernorm(hv + dh, ln1g_ref[...], ln1b_ref[...])
    f = _gelu(_dot(h, win_ref[...]) + bin_ref[...])
    f = _dot(f, wout_ref[...]) + bout_ref[...]
    h2 = _layernorm(h + f, ln2g_ref[...], ln2b_ref[...]) * mv_ref[...]
    hv2_ref[...] = h2
    c2_ref[...] = _dot(h2, w11c_ref[...])


def _edge_body(hv_ref, he_ref, g2_ref,
               w11a_ref, b11_ref, w11b_ref, w12_ref, b12_ref,
               w13_ref, b13_ref, ln3g_ref, ln3b_ref,
               heo_ref):
    a1 = _dot(hv_ref[...], w11a_ref[...]) + b11_ref[...]
    a1e = jnp.reshape(
        jnp.broadcast_to(a1[:, None, :], (TN, K, H)), (TN * K, H))
    he = he_ref[...]
    y = _dot(he, w11b_ref[...]) + g2_ref[...] + a1e
    y = _gelu(y)
    y = _gelu(_dot(y, w12_ref[...]) + b12_ref[...])
    m = _dot(y, w13_ref[...]) + b13_ref[...]
    heo_ref[...] = _layernorm(he + m, ln3g_ref[...], ln3b_ref[...])


def _full(shape):
    return pl.BlockSpec(shape, lambda i: (0,) * len(shape))


def kernel(h_V, h_E, E_idx, mask_V, mask_attend, params):
    p = params
    n = h_V.shape[1]
    nk = n * K
    hv = h_V[0]
    he = jnp.reshape(h_E[0], (nk, H))
    idx = jnp.reshape(E_idx[0], (nk,)).astype(jnp.int32)
    ma = jnp.reshape(mask_attend[0], (nk, 1))
    mv = jnp.reshape(mask_V[0], (n, 1))

    info = plsc.get_sparse_core_info()
    nw = info.num_cores * info.num_subcores
    quanta = nw * CH
    nkpad = ((nk + quanta - 1) // quanta) * quanta
    idx_pad = jnp.concatenate(
        [idx, jnp.zeros((nkpad - nk,), jnp.int32)]) if nkpad != nk else idx

    w1t = p['W1'].T
    w1a, w1b, w1c = w1t[:H], w1t[H:2 * H], w1t[2 * H:]
    w11t = p['W11'].T
    w11a, w11b, w11c = w11t[:H], w11t[H:2 * H], w11t[2 * H:]
    w2, w3 = p['W2'].T, p['W3'].T
    w12, w13 = p['W12'].T, p['W13'].T
    win, wout = p['Win'].T, p['Wout'].T
    row = lambda v: jnp.reshape(v, (1, -1))

    grid = (n // TN,)
    node_spec = pl.BlockSpec((TN, H), lambda i: (i, 0))
    edge_spec = pl.BlockSpec((TN * K, H), lambda i: (i, 0))

    c1 = pl.pallas_call(
        _c1_body,
        grid=(n // 1000,),
        in_specs=[pl.BlockSpec((1000, H), lambda i: (i, 0)), _full((H, H))],
        out_specs=pl.BlockSpec((1000, H), lambda i: (i, 0)),
        out_shape=jax.ShapeDtypeStruct((n, H), jnp.float32),
    )(hv, w1c)

    g1 = _sc_gather(c1, idx_pad)

    hv2, c2 = pl.pallas_call(
        _node_body,
        grid=grid,
        in_specs=[
            node_spec, edge_spec, edge_spec,
            pl.BlockSpec((TN * K, 1), lambda i: (i, 0)),
            pl.BlockSpec((TN, 1), lambda i: (i, 0)),
            _full((H, H)), _full((1, H)), _full((H, H)),
            _full((H, H)), _full((1, H)), _full((H, H)), _full((1, H)),
            _full((H, 4 * H)), _full((1, 4 * H)), _full((4 * H, H)),
            _full((1, H)),
            _full((1, H)), _full((1, H)), _full((1, H)), _full((1, H)),
            _full((H, H)),
        ],
        out_specs=[node_spec, node_spec],
        out_shape=[jax.ShapeDtypeStruct((n, H), jnp.float32),
                   jax.ShapeDtypeStruct((n, H), jnp.float32)],
    )(hv, he, g1, ma, mv,
      w1a, row(p['b1']), w1b, w2, row(p['b2']), w3, row(p['b3']),
      win, row(p['bin']), wout, row(p['bout']),
      row(p['ln1_g']), row(p['ln1_b']), row(p['ln2_g']), row(p['ln2_b']),
      w11c)

    g2 = _sc_gather(c2, idx_pad)

    heo = pl.pallas_call(
        _edge_body,
        grid=grid,
        in_specs=[
            node_spec, edge_spec, edge_spec,
            _full((H, H)), _full((1, H)), _full((H, H)),
            _full((H, H)), _full((1, H)), _full((H, H)), _full((1, H)),
            _full((1, H)), _full((1, H)),
        ],
        out_specs=edge_spec,
        out_shape=jax.ShapeDtypeStruct((nk, H), jnp.float32),
    )(hv2, he, g2,
      w11a, row(p['b11']), w11b, w12, row(p['b12']), w13, row(p['b13']),
      row(p['ln3_g']), row(p['ln3_b']))

    return (hv2[None], jnp.reshape(heo, (1, n, K, H)))


# speedup vs baseline: 5.7847x; 1.0475x over previous
import functools
import math

import jax
import jax.numpy as jnp
from jax import lax
from jax.experimental import pallas as pl
from jax.experimental.pallas import tpu as pltpu
from jax.experimental.pallas import tpu_sc as plsc

H = 128
K = 16
SCALE = 30.0
EPS = 1e-5
TN = 400
CH = 128

_SQRT2 = math.sqrt(2.0)


def _gelu(x):
    return 0.5 * x * (1.0 + lax.erf(x / _SQRT2))


def _layernorm(x, g, b):
    m = jnp.mean(x, axis=-1, keepdims=True)
    v = jnp.mean((x - m) ** 2, axis=-1, keepdims=True)
    return (x - m) * lax.rsqrt(v + EPS) * g + b


def _dot(a, b):
    return jnp.dot(a, b, preferred_element_type=jnp.float32)


def _sc_gather(table, idx):
    nkpad = idx.shape[0]
    info = plsc.get_sparse_core_info()
    nc, ns = info.num_cores, info.num_subcores
    nw = nc * ns
    per_w = nkpad // nw
    assert per_w % CH == 0
    nch = per_w // CH
    mesh = plsc.VectorSubcoreMesh(core_axis_name="c", subcore_axis_name="s")

    nbuf = 4
    assert nch % nbuf == 0

    @functools.partial(
        pl.kernel,
        mesh=mesh,
        out_type=jax.ShapeDtypeStruct((nkpad, H), jnp.float32),
        scratch_types=[
            pltpu.VMEM((per_w,), jnp.int32),
            [pltpu.VMEM((CH, H), jnp.float32) for _ in range(nbuf)],
            [pltpu.SemaphoreType.DMA for _ in range(nbuf)],
            [pltpu.SemaphoreType.DMA for _ in range(nbuf)],
        ],
    )
    def gk(table_hbm, idx_hbm, out_hbm, idx_v, rows, sg, sw):
        wid = lax.axis_index("s") * nc + lax.axis_index("c")
        base = wid * per_w

        pltpu.sync_copy(idx_hbm.at[pl.ds(base, per_w)], idx_v)

        def gather_start(c, b):
            pltpu.async_copy(
                table_hbm.at[idx_v.at[pl.ds(c * CH, CH)]], rows[b], sg[b])

        def gather_wait(c, b):
            pltpu.make_async_copy(
                table_hbm.at[idx_v.at[pl.ds(c * CH, CH)]],
                rows[b], sg[b]).wait()

        def wb_start(c, b):
            pltpu.async_copy(
                rows[b], out_hbm.at[pl.ds(base + c * CH, CH)], sw[b])

        def wb_wait(c, b):
            pltpu.make_async_copy(
                rows[b], out_hbm.at[pl.ds(base + c * CH, CH)], sw[b]).wait()

        gather_start(0, 0)
        gather_start(1, 1)

        def body(g, carry):
            for b in range(nbuf):
                c = g + b
                b2 = (b + 2) % nbuf
                gather_wait(c, b)

                @pl.when(c + 2 < nch)
                def _():
                    @pl.when(c >= 2)
                    def _():
                        wb_wait(c - 2, b2)
                    gather_start(c + 2, b2)

                wb_start(c, b)
            return carry

        lax.fori_loop(0, nch // nbuf, lambda g, c: body(g * nbuf, c), 0)
        for j in range(nbuf):
            c = nch - nbuf + j
            wb_wait(c, c % nbuf)

    return gk(table, idx)


def _node_body(hv_ref, he_ref, g1_ref, ma_ref, mv_ref,
               w1a_ref, b1_ref, w1b_ref, w1c_ref, w2_ref, b2_ref,
               w3_ref, b3_ref,
               win_ref, bin_ref, wout_ref, bout_ref,
               ln1g_ref, ln1b_ref, ln2g_ref, ln2b_ref,
               hv2_ref):
    hv = hv_ref[...]
    a1 = _dot(hv, w1a_ref[...]) + b1_ref[...]
    a1e = jnp.reshape(
        jnp.broadcast_to(a1[:, None, :], (TN, K, H)), (TN * K, H))
    x = (_dot(he_ref[...], w1b_ref[...])
         + _dot(g1_ref[...], w1c_ref[...]) + a1e)
    x = _gelu(x)
    x = _gelu(_dot(x, w2_ref[...]) + b2_ref[...])
    m = _dot(x, w3_ref[...]) + b3_ref[...]
    m = m * ma_ref[...]
    dh = jnp.sum(jnp.reshape(m, (TN, K, H)), axis=1) * (1.0 / SCALE)
    h = _layernorm(hv + dh, ln1g_ref[...], ln1b_ref[...])
    f = _gelu(_dot(h, win_ref[...]) + bin_ref[...])
    f = _dot(f, wout_ref[...]) + bout_ref[...]
    h2 = _layernorm(h + f, ln2g_ref[...], ln2b_ref[...]) * mv_ref[...]
    hv2_ref[...] = h2


def _edge_body(hv_ref, he_ref, g2_ref,
               w11a_ref, b11_ref, w11b_ref, w11c_ref, w12_ref, b12_ref,
               w13_ref, b13_ref, ln3g_ref, ln3b_ref,
               heo_ref):
    a1 = _dot(hv_ref[...], w11a_ref[...]) + b11_ref[...]
    a1e = jnp.reshape(
        jnp.broadcast_to(a1[:, None, :], (TN, K, H)), (TN * K, H))
    he = he_ref[...]
    y = (_dot(he, w11b_ref[...])
         + _dot(g2_ref[...], w11c_ref[...]) + a1e)
    y = _gelu(y)
    y = _gelu(_dot(y, w12_ref[...]) + b12_ref[...])
    m = _dot(y, w13_ref[...]) + b13_ref[...]
    heo_ref[...] = _layernorm(he + m, ln3g_ref[...], ln3b_ref[...])


def _full(shape):
    return pl.BlockSpec(shape, lambda i: (0,) * len(shape))


def kernel(h_V, h_E, E_idx, mask_V, mask_attend, params):
    p = params
    n = h_V.shape[1]
    nk = n * K
    hv = h_V[0]
    he = jnp.reshape(h_E[0], (nk, H))
    idx = jnp.reshape(E_idx[0], (nk,)).astype(jnp.int32)
    ma = jnp.reshape(mask_attend[0], (nk, 1))
    mv = jnp.reshape(mask_V[0], (n, 1))

    info = plsc.get_sparse_core_info()
    nw = info.num_cores * info.num_subcores
    quanta = nw * CH
    nkpad = ((nk + quanta - 1) // quanta) * quanta
    idx_pad = jnp.concatenate(
        [idx, jnp.zeros((nkpad - nk,), jnp.int32)]) if nkpad != nk else idx

    w1t = p['W1'].T
    w1a, w1b, w1c = w1t[:H], w1t[H:2 * H], w1t[2 * H:]
    w11t = p['W11'].T
    w11a, w11b, w11c = w11t[:H], w11t[H:2 * H], w11t[2 * H:]
    w2, w3 = p['W2'].T, p['W3'].T
    w12, w13 = p['W12'].T, p['W13'].T
    win, wout = p['Win'].T, p['Wout'].T
    row = lambda v: jnp.reshape(v, (1, -1))

    grid = (n // TN,)
    node_spec = pl.BlockSpec((TN, H), lambda i: (i, 0))
    edge_spec = pl.BlockSpec((TN * K, H), lambda i: (i, 0))

    g1 = _sc_gather(hv, idx_pad)

    hv2 = pl.pallas_call(
        _node_body,
        grid=grid,
        in_specs=[
            node_spec, edge_spec, edge_spec,
            pl.BlockSpec((TN * K, 1), lambda i: (i, 0)),
            pl.BlockSpec((TN, 1), lambda i: (i, 0)),
            _full((H, H)), _full((1, H)), _full((H, H)), _full((H, H)),
            _full((H, H)), _full((1, H)), _full((H, H)), _full((1, H)),
            _full((H, 4 * H)), _full((1, 4 * H)), _full((4 * H, H)),
            _full((1, H)),
            _full((1, H)), _full((1, H)), _full((1, H)), _full((1, H)),
        ],
        out_specs=node_spec,
        out_shape=jax.ShapeDtypeStruct((n, H), jnp.float32),
    )(hv, he, g1, ma, mv,
      w1a, row(p['b1']), w1b, w1c, w2, row(p['b2']), w3, row(p['b3']),
      win, row(p['bin']), wout, row(p['bout']),
      row(p['ln1_g']), row(p['ln1_b']), row(p['ln2_g']), row(p['ln2_b']))

    g2 = _sc_gather(hv2, idx_pad)

    heo = pl.pallas_call(
        _edge_body,
        grid=grid,
        in_specs=[
            node_spec, edge_spec, edge_spec,
            _full((H, H)), _full((1, H)), _full((H, H)), _full((H, H)),
            _full((H, H)), _full((1, H)), _full((H, H)), _full((1, H)),
            _full((1, H)), _full((1, H)),
        ],
        out_specs=edge_spec,
        out_shape=jax.ShapeDtypeStruct((nk, H), jnp.float32),
    )(hv2, he, g2,
      w11a, row(p['b11']), w11b, w11c, w12, row(p['b12']), w13,
      row(p['b13']), row(p['ln3_g']), row(p['ln3_b']))

    return (hv2[None], jnp.reshape(heo, (1, n, K, H)))
